# baseline (device time: 31191 ns/iter reference)
import jax
import jax.numpy as jnp
from jax import lax
from jax.experimental import pallas as pl
from jax.experimental.pallas import tpu as pltpu

N_CHUNKS = 8


def kernel(x, pi):
    shard_shape = x.shape
    rows = shard_shape[1] // N_CHUNKS

    def body(x_ref, pi_ref, out_ref, stage_ref, comm_ref,
             copy_sem, send_sem, recv_sem):
        my_x = lax.axis_index("x")
        my_y = lax.axis_index("y")
        my_z = lax.axis_index("z")
        partner = 1 - my_x
        tgt_x = pi_ref[my_x]

        barrier = pltpu.get_barrier_semaphore()
        pl.semaphore_signal(
            barrier, inc=1,
            device_id=(partner, my_y, my_z),
            device_id_type=pl.DeviceIdType.MESH,
        )

        def start_fetch(h):
            cp = pltpu.make_async_copy(
                x_ref.at[0, pl.ds(h * rows, rows), :],
                stage_ref.at[h % 2],
                copy_sem.at[h % 2],
            )
            cp.start()
            return cp

        fetches = {0: start_fetch(0)}
        rdmas = []
        for h in range(N_CHUNKS):
            if h + 1 < N_CHUNKS:
                fetches[h + 1] = start_fetch(h + 1)
            fetches[h].wait()
            sl = pl.ds(h * rows, rows)
            comm_ref[0, sl, :] = stage_ref[h % 2].astype(jnp.bfloat16)
            if h == 0:
                pl.semaphore_wait(barrier, 1)
            rdma = pltpu.make_async_remote_copy(
                src_ref=comm_ref.at[0, sl, :],
                dst_ref=out_ref.at[0, sl, :],
                send_sem=send_sem.at[h],
                recv_sem=recv_sem.at[h],
                device_id=(tgt_x, my_y, my_z),
                device_id_type=pl.DeviceIdType.MESH,
            )
            rdma.start()
            rdmas.append(rdma)
        for rdma in rdmas:
            rdma.wait()

    return pl.pallas_call(
        body,
        out_shape=jax.ShapeDtypeStruct(shard_shape, jnp.bfloat16),
        in_specs=[
            pl.BlockSpec(memory_space=pl.ANY),
            pl.BlockSpec(memory_space=pltpu.SMEM),
        ],
        out_specs=pl.BlockSpec(memory_space=pl.ANY),
        scratch_shapes=[
            pltpu.VMEM((2, rows, shard_shape[2]), x.dtype),
            pltpu.VMEM(shard_shape, jnp.bfloat16),
            pltpu.SemaphoreType.DMA((2,)),
            pltpu.SemaphoreType.DMA((N_CHUNKS,)),
            pltpu.SemaphoreType.DMA((N_CHUNKS,)),
        ],
        compiler_params=pltpu.CompilerParams(collective_id=0),
    )(x, pi)


# device time: 24706 ns/iter; 1.2625x vs baseline; 1.2625x over previous
import jax
import jax.numpy as jnp
from jax import lax
from jax.experimental import pallas as pl
from jax.experimental.pallas import tpu as pltpu

Q_ROWS = 256
SUB = 64
N_SUB = Q_ROWS // SUB
N_X = 2 * N_SUB


def kernel(x, pi):
    shard_shape = x.shape
    ncols = shard_shape[2]

    def body(x_ref, pi_ref, out_ref, stage_ref, commx_ref,
             copy_sem, xs_sem, xr_sem, ys_sem, yr_sem, zs_sem, zr_sem):
        my_x = lax.axis_index("x")
        my_y = lax.axis_index("y")
        my_z = lax.axis_index("z")
        tgt_x = pi_ref[my_x]
        p = my_y ^ my_z
        dev_x = (tgt_x, my_y, my_z)
        dev_y = (my_x, 1 - my_y, my_z)
        dev_z = (my_x, my_y, 1 - my_z)

        def row_of(k):
            q = 2 * p + (k % 2)
            return q * Q_ROWS + (k // 2) * SUB

        barrier = pltpu.get_barrier_semaphore()
        for dev in (dev_x, dev_y, dev_z):
            pl.semaphore_signal(
                barrier, inc=1,
                device_id=dev, device_id_type=pl.DeviceIdType.MESH,
            )

        def start_fetch(k):
            cp = pltpu.make_async_copy(
                x_ref.at[0, pl.ds(row_of(k), SUB), :],
                stage_ref.at[k % 2],
                copy_sem.at[k % 2],
            )
            cp.start()
            return cp

        fetches = {0: start_fetch(0)}
        x_rdmas = []
        for k in range(N_X):
            if k + 1 < N_X:
                fetches[k + 1] = start_fetch(k + 1)
            fetches[k].wait()
            commx_ref[pl.ds(k * SUB, SUB), :] = (
                stage_ref[k % 2].astype(jnp.bfloat16))
            if k == 0:
                pl.semaphore_wait(barrier, 3)
            rdma = pltpu.make_async_remote_copy(
                src_ref=commx_ref.at[pl.ds(k * SUB, SUB), :],
                dst_ref=out_ref.at[0, pl.ds(row_of(k), SUB), :],
                send_sem=xs_sem.at[k],
                recv_sem=xr_sem.at[k],
                device_id=dev_x,
                device_id_type=pl.DeviceIdType.MESH,
            )
            rdma.start()
            x_rdmas.append(rdma)

        face_rdmas = []
        for k in range(N_X):
            x_rdmas[k].wait_recv()
            sems, dev = ((ys_sem, dev_y) if k % 2 == 0
                         else (zs_sem, dev_z))
            rsems = yr_sem if k % 2 == 0 else zr_sem
            fwd = pltpu.make_async_remote_copy(
                src_ref=out_ref.at[0, pl.ds(row_of(k), SUB), :],
                dst_ref=out_ref.at[0, pl.ds(row_of(k), SUB), :],
                send_sem=sems.at[k // 2],
                recv_sem=rsems.at[k // 2],
                device_id=dev,
                device_id_type=pl.DeviceIdType.MESH,
            )
            fwd.start()
            face_rdmas.append(fwd)

        for rdma in x_rdmas:
            rdma.wait_send()
        for fwd in face_rdmas:
            fwd.wait()

    return pl.pallas_call(
        body,
        out_shape=jax.ShapeDtypeStruct(shard_shape, jnp.bfloat16),
        in_specs=[
            pl.BlockSpec(memory_space=pl.ANY),
            pl.BlockSpec(memory_space=pltpu.SMEM),
        ],
        out_specs=pl.BlockSpec(memory_space=pltpu.VMEM),
        scratch_shapes=[
            pltpu.VMEM((2, SUB, ncols), x.dtype),
            pltpu.VMEM((N_X * SUB, ncols), jnp.bfloat16),
            pltpu.SemaphoreType.DMA((2,)),
            pltpu.SemaphoreType.DMA((N_X,)),
            pltpu.SemaphoreType.DMA((N_X,)),
            pltpu.SemaphoreType.DMA((N_SUB,)),
            pltpu.SemaphoreType.DMA((N_SUB,)),
            pltpu.SemaphoreType.DMA((N_SUB,)),
            pltpu.SemaphoreType.DMA((N_SUB,)),
        ],
        compiler_params=pltpu.CompilerParams(collective_id=0),
    )(x, pi)


# device time: 21949 ns/iter; 1.4211x vs baseline; 1.1256x over previous
import jax
import jax.numpy as jnp
from jax import lax
from jax.experimental import pallas as pl
from jax.experimental.pallas import tpu as pltpu

Q_ROWS = 256
SUB = 64
N_SUB = Q_ROWS // SUB
N_X = N_SUB + 2


def kernel(x, pi):
    shard_shape = x.shape
    ncols = shard_shape[2]

    def body(x_ref, pi_ref, out_ref, stage_ref, commx_ref,
             copy_sem, xs_sem, xr_sem, ys_sem, yr_sem, zs_sem, zr_sem):
        my_x = lax.axis_index("x")
        my_y = lax.axis_index("y")
        my_z = lax.axis_index("z")
        tgt_x = pi_ref[my_x]
        dev_x = (tgt_x, my_y, my_z)
        dev_y = (my_x, 1 - my_y, my_z)
        dev_z = (my_x, my_y, 1 - my_z)

        q_me = 2 * my_y + my_z
        q_diag = 2 * (1 - my_y) + (1 - my_z)
        q_zn = 2 * my_y + (1 - my_z)

        def xrow(k):
            if k < N_SUB:
                return q_me * Q_ROWS + k * SUB
            return q_diag * Q_ROWS + (k - N_SUB) * SUB

        barrier = pltpu.get_barrier_semaphore()
        for dev in (dev_x, dev_y, dev_z):
            pl.semaphore_signal(
                barrier, inc=1,
                device_id=dev, device_id_type=pl.DeviceIdType.MESH,
            )

        def start_fetch(k):
            cp = pltpu.make_async_copy(
                x_ref.at[0, pl.ds(xrow(k), SUB), :],
                stage_ref.at[k % 2],
                copy_sem.at[k % 2],
            )
            cp.start()
            return cp

        fetches = {0: start_fetch(0)}
        x_rdmas = []
        for k in range(N_X):
            if k + 1 < N_X:
                fetches[k + 1] = start_fetch(k + 1)
            fetches[k].wait()
            commx_ref[pl.ds(k * SUB, SUB), :] = (
                stage_ref[k % 2].astype(jnp.bfloat16))
            if k == 0:
                pl.semaphore_wait(barrier, 3)
            rdma = pltpu.make_async_remote_copy(
                src_ref=commx_ref.at[pl.ds(k * SUB, SUB), :],
                dst_ref=out_ref.at[0, pl.ds(xrow(k), SUB), :],
                send_sem=xs_sem.at[k],
                recv_sem=xr_sem.at[k],
                device_id=dev_x,
                device_id_type=pl.DeviceIdType.MESH,
            )
            rdma.start()
            x_rdmas.append(rdma)

        def fwd(row, ssem, rsem, dev):
            r = pltpu.make_async_remote_copy(
                src_ref=out_ref.at[0, pl.ds(row, SUB), :],
                dst_ref=out_ref.at[0, pl.ds(row, SUB), :],
                send_sem=ssem,
                recv_sem=rsem,
                device_id=dev,
                device_id_type=pl.DeviceIdType.MESH,
            )
            r.start()
            return r

        y_rdmas = []
        z_rdmas = []
        for k in range(N_SUB):
            x_rdmas[k].wait_recv()
            row = q_me * Q_ROWS + k * SUB
            y_rdmas.append(fwd(row, ys_sem.at[k], yr_sem.at[k], dev_y))
            z_rdmas.append(fwd(row, zs_sem.at[k], zr_sem.at[k], dev_z))

        relay_rdmas = []
        for j in (2, 3):
            z_rdmas[j].wait_recv()
            row = q_zn * Q_ROWS + j * SUB
            relay_rdmas.append(
                fwd(row, ys_sem.at[N_SUB + j - 2],
                    yr_sem.at[N_SUB + j - 2], dev_y))

        for k in range(N_X):
            x_rdmas[k].wait_send()
            if k >= N_SUB:
                x_rdmas[k].wait_recv()
        for r in y_rdmas:
            r.wait()
        for j, r in enumerate(z_rdmas):
            r.wait_send()
            if j < 2:
                r.wait_recv()
        for r in relay_rdmas:
            r.wait()

    return pl.pallas_call(
        body,
        out_shape=jax.ShapeDtypeStruct(shard_shape, jnp.bfloat16),
        in_specs=[
            pl.BlockSpec(memory_space=pl.ANY),
            pl.BlockSpec(memory_space=pltpu.SMEM),
        ],
        out_specs=pl.BlockSpec(memory_space=pltpu.VMEM),
        scratch_shapes=[
            pltpu.VMEM((2, SUB, ncols), x.dtype),
            pltpu.VMEM((N_X * SUB, ncols), jnp.bfloat16),
            pltpu.SemaphoreType.DMA((2,)),
            pltpu.SemaphoreType.DMA((N_X,)),
            pltpu.SemaphoreType.DMA((N_X,)),
            pltpu.SemaphoreType.DMA((N_SUB + 2,)),
            pltpu.SemaphoreType.DMA((N_SUB + 2,)),
            pltpu.SemaphoreType.DMA((N_SUB,)),
            pltpu.SemaphoreType.DMA((N_SUB,)),
        ],
        compiler_params=pltpu.CompilerParams(collective_id=0),
    )(x, pi)


# device time: 20167 ns/iter; 1.5466x vs baseline; 1.0884x over previous
import jax
import jax.numpy as jnp
from jax import lax
from jax.experimental import pallas as pl
from jax.experimental.pallas import tpu as pltpu

Q_ROWS = 256
SUB = 32
NQ = Q_ROWS // SUB
N_X = NQ + NQ // 2
XQ_ORDER = [4, 5, 6, 7, 0, 1, 2, 3]


def kernel(x, pi):
    shard_shape = x.shape
    ncols = shard_shape[2]

    def body(x_ref, pi_ref, out_ref, stage_ref, commx_ref,
             copy_sem, xs_sem, xr_sem, ys_sem, yr_sem, zs_sem, zr_sem):
        my_x = lax.axis_index("x")
        my_y = lax.axis_index("y")
        my_z = lax.axis_index("z")
        tgt_x = pi_ref[my_x]
        dev_x = (tgt_x, my_y, my_z)
        dev_y = (my_x, 1 - my_y, my_z)
        dev_z = (my_x, my_y, 1 - my_z)

        q_me = 2 * my_y + my_z
        q_diag = 2 * (1 - my_y) + (1 - my_z)
        q_yn = 2 * (1 - my_y) + my_z
        q_zn = 2 * my_y + (1 - my_z)

        def xrow(k):
            if k < NQ:
                return q_me * Q_ROWS + XQ_ORDER[k] * SUB
            return q_diag * Q_ROWS + (k - NQ) * SUB

        barrier = pltpu.get_barrier_semaphore()
        for dev in (dev_x, dev_y, dev_z):
            pl.semaphore_signal(
                barrier, inc=1,
                device_id=dev, device_id_type=pl.DeviceIdType.MESH,
            )

        def start_fetch(k):
            cp = pltpu.make_async_copy(
                x_ref.at[0, pl.ds(xrow(k), SUB), :],
                stage_ref.at[k % 2],
                copy_sem.at[k % 2],
            )
            cp.start()
            return cp

        fetches = {0: start_fetch(0)}
        x_rdmas = []
        for k in range(N_X):
            if k + 1 < N_X:
                fetches[k + 1] = start_fetch(k + 1)
            fetches[k].wait()
            commx_ref[pl.ds(k * SUB, SUB), :] = (
                stage_ref[k % 2].astype(jnp.bfloat16))
            if k == 0:
                pl.semaphore_wait(barrier, 3)
            rdma = pltpu.make_async_remote_copy(
                src_ref=commx_ref.at[pl.ds(k * SUB, SUB), :],
                dst_ref=out_ref.at[0, pl.ds(xrow(k), SUB), :],
                send_sem=xs_sem.at[k],
                recv_sem=xr_sem.at[k],
                device_id=dev_x,
                device_id_type=pl.DeviceIdType.MESH,
            )
            rdma.start()
            x_rdmas.append(rdma)

        def fwd(row, ssem, rsem, dev):
            r = pltpu.make_async_remote_copy(
                src_ref=out_ref.at[0, pl.ds(row, SUB), :],
                dst_ref=out_ref.at[0, pl.ds(row, SUB), :],
                send_sem=ssem,
                recv_sem=rsem,
                device_id=dev,
                device_id_type=pl.DeviceIdType.MESH,
            )
            r.start()
            return r

        y_rdmas = []
        z_rdmas = []
        y_recv_waited = set()
        z_recv_waited = set()
        for k in range(NQ):
            x_rdmas[k].wait_recv()
            row = q_me * Q_ROWS + XQ_ORDER[k] * SUB
            y_rdmas.append(fwd(row, ys_sem.at[k], yr_sem.at[k], dev_y))
            z_rdmas.append(fwd(row, zs_sem.at[k], zr_sem.at[k], dev_z))

        relay_rdmas = []
        for i, pos in enumerate((0, 1)):
            z_rdmas[pos].wait_recv()
            z_recv_waited.add(pos)
            row = q_zn * Q_ROWS + (4 + i) * SUB
            relay_rdmas.append(
                fwd(row, ys_sem.at[NQ + i], yr_sem.at[NQ + i], dev_y))
        for i, pos in enumerate((2, 3)):
            y_rdmas[pos].wait_recv()
            y_recv_waited.add(pos)
            row = q_yn * Q_ROWS + (6 + i) * SUB
            relay_rdmas.append(
                fwd(row, zs_sem.at[NQ + i], zr_sem.at[NQ + i], dev_z))

        for k in range(N_X):
            x_rdmas[k].wait_send()
            if k >= NQ:
                x_rdmas[k].wait_recv()
        for k in range(NQ):
            y_rdmas[k].wait_send()
            if k not in y_recv_waited:
                y_rdmas[k].wait_recv()
            z_rdmas[k].wait_send()
            if k not in z_recv_waited:
                z_rdmas[k].wait_recv()
        for r in relay_rdmas:
            r.wait()

    return pl.pallas_call(
        body,
        out_shape=jax.ShapeDtypeStruct(shard_shape, jnp.bfloat16),
        in_specs=[
            pl.BlockSpec(memory_space=pl.ANY),
            pl.BlockSpec(memory_space=pltpu.SMEM),
        ],
        out_specs=pl.BlockSpec(memory_space=pltpu.VMEM),
        scratch_shapes=[
            pltpu.VMEM((2, SUB, ncols), x.dtype),
            pltpu.VMEM((N_X * SUB, ncols), jnp.bfloat16),
            pltpu.SemaphoreType.DMA((2,)),
            pltpu.SemaphoreType.DMA((N_X,)),
            pltpu.SemaphoreType.DMA((N_X,)),
            pltpu.SemaphoreType.DMA((NQ + 2,)),
            pltpu.SemaphoreType.DMA((NQ + 2,)),
            pltpu.SemaphoreType.DMA((NQ + 2,)),
            pltpu.SemaphoreType.DMA((NQ + 2,)),
        ],
        compiler_params=pltpu.CompilerParams(collective_id=0),
    )(x, pi)
